# baseline (speedup 1.0000x reference)
import functools
import jax
import jax.numpy as jnp
from jax import lax
from jax.experimental import pallas as pl
from jax.experimental.pallas import tpu as pltpu
from jax.experimental.pallas import tpu_sc as plsc

_K = 64
_NC = 2
_NS = 16
_L = 16


def _tc_body(ck_ref, out_ref):
    int_min = jnp.int32(-2147483648)
    x = ck_ref[...]
    n = x.shape[-1]
    i = lax.bitcast_convert_type(x, jnp.int32)
    key = jnp.where(i < 0, i ^ jnp.int32(0x7FFFFFFF), i)

    cnt_pos = jnp.sum((key >= 0).astype(jnp.int32), axis=1, keepdims=True)
    t = jnp.where(cnt_pos >= _K, jnp.int32(0), int_min)

    def step(b, t):
        t_try = t | (jnp.int32(1) << (30 - b))
        cnt = jnp.sum((key >= t_try).astype(jnp.int32), axis=1, keepdims=True)
        return jnp.where(cnt >= _K, t_try, t)

    t = lax.fori_loop(0, 31, step, t, unroll=True)

    gt = key > t
    cnt_gt = jnp.sum(gt.astype(jnp.int32), axis=1, keepdims=True)
    r = _K - cnt_gt

    idx = lax.broadcasted_iota(jnp.int32, x.shape, 1)
    key2 = jnp.where(key == t, jnp.int32(n - 1) - idx, jnp.int32(-1))

    def step2(b, t2):
        t_try = t2 | (jnp.int32(1) << (12 - b))
        cnt = jnp.sum((key2 >= t_try).astype(jnp.int32), axis=1, keepdims=True)
        return jnp.where(cnt >= r, t_try, t2)

    t2 = lax.fori_loop(0, 13, step2, jnp.zeros_like(t), unroll=True)

    out_ref[...] = (gt | (key2 >= t2)).astype(jnp.float32)


def _tc_kernel(ck):
    return pl.pallas_call(
        _tc_body,
        out_shape=jax.ShapeDtypeStruct(ck.shape, jnp.float32),
    )(ck)


def _descent(count_fn, kk, nbits, t0):
    def dstep(b, t):
        t_try = t | (jnp.int32(1) << (nbits - 1 - b))
        return jnp.where(count_fn(t_try) >= kk, t_try, t)
    return lax.fori_loop(0, nbits, dstep, t0)


def _sc_body(n, rpw, ck_hbm, out_hbm, x_v, key_v, candk_v, candi_v, m_v, out_v):
    nsl = n // _L
    nmsl = nsl // _L
    int_min = jnp.int32(-2147483648)
    zero16 = jnp.zeros((_L,), jnp.int32)
    lane = lax.iota(jnp.int32, _L)

    wid = lax.axis_index("s") * _NC + lax.axis_index("c")

    def do_row(rr, _):
        row = wid * rpw + rr
        pltpu.sync_copy(ck_hbm.at[row], x_v)

        def k_group(g, _):
            def k_step(q, acc):
                j = g * _L + q
                v = x_v[pl.ds(j * _L, _L)]
                i = lax.bitcast_convert_type(v, jnp.int32)
                k = jnp.where(i < 0, i ^ jnp.int32(0x7FFFFFFF), i)
                key_v[pl.ds(j * _L, _L)] = k
                return jnp.maximum(acc, k)
            acc = lax.fori_loop(0, _L, k_step,
                                jnp.full((_L,), int_min, jnp.int32),
                                unroll=True)
            m_v[pl.ds(g * _L, _L)] = acc
            return 0
        lax.fori_loop(0, nmsl, k_group, 0)

        def m_count(t):
            def cstep(j, acc):
                v = m_v[pl.ds(j * _L, _L)]
                return acc + jnp.where(v >= t, jnp.int32(1), jnp.int32(0))
            return jnp.sum(lax.fori_loop(0, nmsl, cstep, zero16, unroll=8))

        t0 = jnp.where(m_count(jnp.int32(0)) >= _K, jnp.int32(0), int_min)
        m64 = _descent(m_count, _K, 31, t0)

        def c_step(j, off):
            v = key_v[pl.ds(j * _L, _L)]
            msk = v >= m64
            plsc.store_compressed(candk_v.at[pl.ds(off, _L)], v, mask=msk)
            plsc.store_compressed(candi_v.at[pl.ds(off, _L)], lane + j * _L,
                                  mask=msk)
            return off + jnp.sum(jnp.where(msk, jnp.int32(1), jnp.int32(0)))
        ncand = lax.fori_loop(0, nsl, c_step, jnp.int32(0), unroll=8)
        ncsl = (ncand + _L - 1) // _L

        def cand_count(t):
            def cstep(j, acc):
                v = candk_v[pl.ds(j * _L, _L)]
                valid = (j * _L + lane) < ncand
                return acc + jnp.where(valid & (v >= t), jnp.int32(1),
                                       jnp.int32(0))
            return jnp.sum(lax.fori_loop(0, ncsl, cstep, zero16))

        t0c = jnp.where(cand_count(jnp.int32(0)) >= _K, jnp.int32(0), int_min)
        t_star = _descent(cand_count, _K, 31, t0c)

        def gt_count(j, acc):
            v = candk_v[pl.ds(j * _L, _L)]
            valid = (j * _L + lane) < ncand
            return acc + jnp.where(valid & (v > t_star), jnp.int32(1),
                                   jnp.int32(0))
        r = _K - jnp.sum(lax.fori_loop(0, ncsl, gt_count, zero16))

        def tie_count(t2):
            def cstep(j, acc):
                v = candk_v[pl.ds(j * _L, _L)]
                ii = candi_v[pl.ds(j * _L, _L)]
                valid = (j * _L + lane) < ncand
                k2 = jnp.where(valid & (v == t_star), jnp.int32(n - 1) - ii,
                               jnp.int32(-1))
                return acc + jnp.where(k2 >= t2, jnp.int32(1), jnp.int32(0))
            return jnp.sum(lax.fori_loop(0, ncsl, cstep, zero16))

        t2 = _descent(tie_count, r, 13, jnp.int32(0))

        def o_step(j, _):
            v = key_v[pl.ds(j * _L, _L)]
            ii = lane + j * _L
            k2 = jnp.where(v == t_star, jnp.int32(n - 1) - ii, jnp.int32(-1))
            sel = (v > t_star) | (k2 >= t2)
            out_v[pl.ds(j * _L, _L)] = jnp.where(sel, jnp.float32(1.0),
                                                 jnp.float32(0.0))
            return 0
        lax.fori_loop(0, nsl, o_step, 0, unroll=8)
        pltpu.sync_copy(out_v, out_hbm.at[row])
        return 0

    lax.fori_loop(0, rpw, do_row, 0)


@functools.lru_cache(maxsize=None)
def _make_sc_topk_mask(nrows, n):
    assert nrows % (_NC * _NS) == 0
    rpw = nrows // (_NC * _NS)
    mesh = plsc.VectorSubcoreMesh(core_axis_name="c", subcore_axis_name="s",
                                  num_cores=_NC, num_subcores=_NS)
    return pl.kernel(
        functools.partial(_sc_body, n, rpw),
        out_type=jax.ShapeDtypeStruct((nrows, n), jnp.float32),
        mesh=mesh,
        scratch_types=[
            pltpu.VMEM((n,), jnp.float32),
            pltpu.VMEM((n,), jnp.int32),
            pltpu.VMEM((n,), jnp.int32),
            pltpu.VMEM((n,), jnp.int32),
            pltpu.VMEM((n // _L,), jnp.int32),
            pltpu.VMEM((n,), jnp.float32),
        ],
        compiler_params=pltpu.CompilerParams(needs_layout_passes=False),
    )


_SC_ROWS = 128


def kernel(ck):
    nrows = ck.shape[0]
    sc_rows = _SC_ROWS if nrows > _SC_ROWS else nrows
    if sc_rows == nrows:
        return _make_sc_topk_mask(nrows, ck.shape[1])(ck)
    sc_out = _make_sc_topk_mask(sc_rows, ck.shape[1])(ck[nrows - sc_rows:])
    tc_out = _tc_kernel(ck[: nrows - sc_rows])
    return jnp.concatenate([tc_out, sc_out], axis=0)


# speedup vs baseline: 1.4305x; 1.4305x over previous
import functools
import jax
import jax.numpy as jnp
from jax import lax
from jax.experimental import pallas as pl
from jax.experimental.pallas import tpu as pltpu
from jax.experimental.pallas import tpu_sc as plsc

_K = 64
_NC = 2
_NS = 16
_L = 16


def _tc_body(ck_ref, out_ref):
    int_min = jnp.int32(-2147483648)
    x = ck_ref[...]
    n = x.shape[-1]
    i = lax.bitcast_convert_type(x, jnp.int32)
    key = jnp.where(i < 0, i ^ jnp.int32(0x7FFFFFFF), i)

    cnt_pos = jnp.sum((key >= 0).astype(jnp.int32), axis=1, keepdims=True)
    t = jnp.where(cnt_pos >= _K, jnp.int32(0), int_min)

    def step(b, t):
        t_try = t | (jnp.int32(1) << (30 - b))
        cnt = jnp.sum((key >= t_try).astype(jnp.int32), axis=1, keepdims=True)
        return jnp.where(cnt >= _K, t_try, t)

    t = lax.fori_loop(0, 31, step, t, unroll=True)

    gt = key > t
    cnt_gt = jnp.sum(gt.astype(jnp.int32), axis=1, keepdims=True)
    r = _K - cnt_gt

    idx = lax.broadcasted_iota(jnp.int32, x.shape, 1)
    key2 = jnp.where(key == t, jnp.int32(n - 1) - idx, jnp.int32(-1))

    def step2(b, t2):
        t_try = t2 | (jnp.int32(1) << (12 - b))
        cnt = jnp.sum((key2 >= t_try).astype(jnp.int32), axis=1, keepdims=True)
        return jnp.where(cnt >= r, t_try, t2)

    t2 = lax.fori_loop(0, 13, step2, jnp.zeros_like(t), unroll=True)

    out_ref[...] = (gt | (key2 >= t2)).astype(jnp.float32)


def _tc_kernel(ck, nrows_out=None, blk=16):
    n = ck.shape[1]
    if nrows_out is None:
        nrows_out = ck.shape[0]
    return pl.pallas_call(
        _tc_body,
        grid=(nrows_out // blk,),
        in_specs=[pl.BlockSpec((blk, n), lambda i: (i, 0))],
        out_specs=pl.BlockSpec((blk, n), lambda i: (i, 0)),
        out_shape=jax.ShapeDtypeStruct((ck.shape[0], n), jnp.float32),
    )(ck)


def _descent(count_fn, kk, nbits, t0):
    def dstep(b, t):
        t_try = t | (jnp.int32(1) << (nbits - 1 - b))
        return jnp.where(count_fn(t_try) >= kk, t_try, t)
    return lax.fori_loop(0, nbits, dstep, t0)


def _sc_body(n, rpw, base, ck_hbm, out_hbm, x_v, key_v, candk_v, candi_v, m_v,
             out_v):
    nsl = n // _L
    nmsl = nsl // _L
    int_min = jnp.int32(-2147483648)
    zero16 = jnp.zeros((_L,), jnp.int32)
    lane = lax.iota(jnp.int32, _L)

    wid = lax.axis_index("s") * _NC + lax.axis_index("c")

    def do_row(rr, _):
        row = wid * rpw + rr
        pltpu.sync_copy(ck_hbm.at[base + row], x_v)

        def k_group(g, _):
            def k_step(q, acc):
                j = g * _L + q
                v = x_v[pl.ds(j * _L, _L)]
                i = lax.bitcast_convert_type(v, jnp.int32)
                k = jnp.where(i < 0, i ^ jnp.int32(0x7FFFFFFF), i)
                key_v[pl.ds(j * _L, _L)] = k
                return jnp.maximum(acc, k)
            acc = lax.fori_loop(0, _L, k_step,
                                jnp.full((_L,), int_min, jnp.int32),
                                unroll=True)
            m_v[pl.ds(g * _L, _L)] = acc
            return 0
        lax.fori_loop(0, nmsl, k_group, 0)

        def m_count(t):
            def cstep(j, acc):
                v = m_v[pl.ds(j * _L, _L)]
                return acc + jnp.where(v >= t, jnp.int32(1), jnp.int32(0))
            return jnp.sum(lax.fori_loop(0, nmsl, cstep, zero16, unroll=8))

        t0 = jnp.where(m_count(jnp.int32(0)) >= _K, jnp.int32(0), int_min)
        m64 = _descent(m_count, _K, 31, t0)

        def c_step(j, off):
            v = key_v[pl.ds(j * _L, _L)]
            msk = v >= m64
            plsc.store_compressed(candk_v.at[pl.ds(off, _L)], v, mask=msk)
            plsc.store_compressed(candi_v.at[pl.ds(off, _L)], lane + j * _L,
                                  mask=msk)
            return off + jnp.sum(jnp.where(msk, jnp.int32(1), jnp.int32(0)))
        ncand = lax.fori_loop(0, nsl, c_step, jnp.int32(0), unroll=8)
        ncsl = (ncand + _L - 1) // _L

        def cand_count(t):
            def cstep(j, acc):
                v = candk_v[pl.ds(j * _L, _L)]
                valid = (j * _L + lane) < ncand
                return acc + jnp.where(valid & (v >= t), jnp.int32(1),
                                       jnp.int32(0))
            return jnp.sum(lax.fori_loop(0, ncsl, cstep, zero16))

        t0c = jnp.where(cand_count(jnp.int32(0)) >= _K, jnp.int32(0), int_min)
        t_star = _descent(cand_count, _K, 31, t0c)

        def gt_count(j, acc):
            v = candk_v[pl.ds(j * _L, _L)]
            valid = (j * _L + lane) < ncand
            return acc + jnp.where(valid & (v > t_star), jnp.int32(1),
                                   jnp.int32(0))
        r = _K - jnp.sum(lax.fori_loop(0, ncsl, gt_count, zero16))

        def tie_count(t2):
            def cstep(j, acc):
                v = candk_v[pl.ds(j * _L, _L)]
                ii = candi_v[pl.ds(j * _L, _L)]
                valid = (j * _L + lane) < ncand
                k2 = jnp.where(valid & (v == t_star), jnp.int32(n - 1) - ii,
                               jnp.int32(-1))
                return acc + jnp.where(k2 >= t2, jnp.int32(1), jnp.int32(0))
            return jnp.sum(lax.fori_loop(0, ncsl, cstep, zero16))

        t2 = _descent(tie_count, r, 13, jnp.int32(0))

        def o_step(j, _):
            v = key_v[pl.ds(j * _L, _L)]
            ii = lane + j * _L
            k2 = jnp.where(v == t_star, jnp.int32(n - 1) - ii, jnp.int32(-1))
            sel = (v > t_star) | (k2 >= t2)
            out_v[pl.ds(j * _L, _L)] = jnp.where(sel, jnp.float32(1.0),
                                                 jnp.float32(0.0))
            return 0
        lax.fori_loop(0, nsl, o_step, 0, unroll=8)
        pltpu.sync_copy(out_v, out_hbm.at[row])
        return 0

    lax.fori_loop(0, rpw, do_row, 0)


@functools.lru_cache(maxsize=None)
def _make_sc_topk_mask(nrows, n, base=0):
    assert nrows % (_NC * _NS) == 0
    rpw = nrows // (_NC * _NS)
    mesh = plsc.VectorSubcoreMesh(core_axis_name="c", subcore_axis_name="s",
                                  num_cores=_NC, num_subcores=_NS)
    return pl.kernel(
        functools.partial(_sc_body, n, rpw, base),
        out_type=jax.ShapeDtypeStruct((nrows, n), jnp.float32),
        mesh=mesh,
        scratch_types=[
            pltpu.VMEM((n,), jnp.float32),
            pltpu.VMEM((n,), jnp.int32),
            pltpu.VMEM((n,), jnp.int32),
            pltpu.VMEM((n,), jnp.int32),
            pltpu.VMEM((n // _L,), jnp.int32),
            pltpu.VMEM((n,), jnp.float32),
        ],
        compiler_params=pltpu.CompilerParams(needs_layout_passes=False),
    )


_SC_ROWS = 32


def kernel(ck):
    nrows = ck.shape[0]
    sc_rows = _SC_ROWS if nrows > _SC_ROWS else nrows
    if sc_rows == nrows:
        return _make_sc_topk_mask(nrows, ck.shape[1])(ck)
    tc_rows = nrows - sc_rows
    sc_out = _make_sc_topk_mask(sc_rows, ck.shape[1], base=tc_rows)(ck)
    tc_out = _tc_kernel(ck, nrows_out=tc_rows)
    return lax.dynamic_update_slice(tc_out, sc_out, (tc_rows, 0))


# speedup vs baseline: 1.8361x; 1.2835x over previous
import functools
import jax
import jax.numpy as jnp
from jax import lax
from jax.experimental import pallas as pl
from jax.experimental.pallas import tpu as pltpu
from jax.experimental.pallas import tpu_sc as plsc

_K = 64
_NC = 2
_NS = 16
_L = 16


def _tc_body(ck_ref, out_ref):
    int_min = jnp.int32(-2147483648)
    x = ck_ref[...]
    n = x.shape[-1]
    i = lax.bitcast_convert_type(x, jnp.int32)
    key = jnp.where(i < 0, i ^ jnp.int32(0x7FFFFFFF), i)

    cnt_pos = jnp.sum((key >= 0).astype(jnp.int32), axis=1, keepdims=True)
    t = jnp.where(cnt_pos >= _K, jnp.int32(0), int_min)

    def step(b, t):
        t_try = t | (jnp.int32(1) << (30 - b))
        cnt = jnp.sum((key >= t_try).astype(jnp.int32), axis=1, keepdims=True)
        return jnp.where(cnt >= _K, t_try, t)

    t = lax.fori_loop(0, 31, step, t, unroll=True)

    gt = key > t
    cnt_gt = jnp.sum(gt.astype(jnp.int32), axis=1, keepdims=True)
    r = _K - cnt_gt

    idx = lax.broadcasted_iota(jnp.int32, x.shape, 1)
    key2 = jnp.where(key == t, jnp.int32(n - 1) - idx, jnp.int32(-1))

    def step2(b, t2):
        t_try = t2 | (jnp.int32(1) << (12 - b))
        cnt = jnp.sum((key2 >= t_try).astype(jnp.int32), axis=1, keepdims=True)
        return jnp.where(cnt >= r, t_try, t2)

    t2 = lax.fori_loop(0, 13, step2, jnp.zeros_like(t), unroll=True)

    out_ref[...] = (gt | (key2 >= t2)).astype(jnp.float32)


def _tc_kernel(ck, nrows_out=None):
    n = ck.shape[1]
    if nrows_out is None:
        nrows_out = ck.shape[0]
    return pl.pallas_call(
        _tc_body,
        grid=(1,),
        in_specs=[pl.BlockSpec((nrows_out, n), lambda i: (0, 0))],
        out_specs=pl.BlockSpec((nrows_out, n), lambda i: (0, 0)),
        out_shape=jax.ShapeDtypeStruct((ck.shape[0], n), jnp.float32),
    )(ck)


def _descent(count_fn, kk, nbits, t0, shift_base=None):
    if shift_base is None:
        shift_base = nbits - 1
    def dstep(b, t):
        t_try = t | (jnp.int32(1) << (shift_base - b))
        return jnp.where(count_fn(t_try) >= kk, t_try, t)
    return lax.fori_loop(0, nbits, dstep, t0)


def _sc_body(n, rpw, base, ck_hbm, out_hbm, x_v, key_v, candk_v, candi_v, m_v,
             out_v):
    nsl = n // _L
    nmsl = nsl // _L
    int_min = jnp.int32(-2147483648)
    zero16 = jnp.zeros((_L,), jnp.int32)
    lane = lax.iota(jnp.int32, _L)

    wid = lax.axis_index("s") * _NC + lax.axis_index("c")

    def do_row(rr, _):
        row = wid * rpw + rr
        pltpu.sync_copy(ck_hbm.at[base + row], x_v)

        def k_group(g, _):
            def k_step(q, acc):
                j = g * _L + q
                v = x_v[pl.ds(j * _L, _L)]
                i = lax.bitcast_convert_type(v, jnp.int32)
                k = jnp.where(i < 0, i ^ jnp.int32(0x7FFFFFFF), i)
                key_v[pl.ds(j * _L, _L)] = k
                return jnp.maximum(acc, k)
            acc = lax.fori_loop(0, _L, k_step,
                                jnp.full((_L,), int_min, jnp.int32),
                                unroll=True)
            m_v[pl.ds(g * _L, _L)] = acc
            return 0
        lax.fori_loop(0, nmsl, k_group, 0)

        def m_count(t):
            def cstep(j, acc):
                v = m_v[pl.ds(j * _L, _L)]
                return acc + jnp.where(v >= t, jnp.int32(1), jnp.int32(0))
            return jnp.sum(lax.fori_loop(0, nmsl, cstep, zero16, unroll=8))

        t0 = jnp.where(m_count(jnp.int32(0)) >= _K, jnp.int32(0), int_min)
        pivot = _descent(m_count, _K, 15, t0, shift_base=30)

        def c_step(j, off):
            v = key_v[pl.ds(j * _L, _L)]
            msk = v >= pivot
            out_v[pl.ds(j * _L, _L)] = jnp.where(msk, jnp.float32(1.0),
                                                 jnp.float32(0.0))
            plsc.store_compressed(candk_v.at[pl.ds(off, _L)], v, mask=msk)
            plsc.store_compressed(candi_v.at[pl.ds(off, _L)], lane + j * _L,
                                  mask=msk)
            return off + jnp.sum(jnp.where(msk, jnp.int32(1), jnp.int32(0)))
        ncand = lax.fori_loop(0, nsl, c_step, jnp.int32(0), unroll=8)
        ncsl = (ncand + _L - 1) // _L

        def cand_count(t):
            def cstep(j, acc):
                v = candk_v[pl.ds(j * _L, _L)]
                valid = (j * _L + lane) < ncand
                return acc + jnp.where(valid & (v >= t), jnp.int32(1),
                                       jnp.int32(0))
            return jnp.sum(lax.fori_loop(0, ncsl, cstep, zero16))

        t0c = jnp.where(cand_count(jnp.int32(0)) >= _K, jnp.int32(0), int_min)
        t_star = _descent(cand_count, _K, 31, t0c)

        def gt_count(j, acc):
            v = candk_v[pl.ds(j * _L, _L)]
            valid = (j * _L + lane) < ncand
            return acc + jnp.where(valid & (v > t_star), jnp.int32(1),
                                   jnp.int32(0))
        r = _K - jnp.sum(lax.fori_loop(0, ncsl, gt_count, zero16))

        def tie_count(t2):
            def cstep(j, acc):
                v = candk_v[pl.ds(j * _L, _L)]
                ii = candi_v[pl.ds(j * _L, _L)]
                valid = (j * _L + lane) < ncand
                k2 = jnp.where(valid & (v == t_star), jnp.int32(n - 1) - ii,
                               jnp.int32(-1))
                return acc + jnp.where(k2 >= t2, jnp.int32(1), jnp.int32(0))
            return jnp.sum(lax.fori_loop(0, ncsl, cstep, zero16))

        t2 = _descent(tie_count, r, 13, jnp.int32(0))

        def f_step(j, _):
            v = candk_v[pl.ds(j * _L, _L)]
            ii = candi_v[pl.ds(j * _L, _L)]
            valid = (j * _L + lane) < ncand
            k2 = jnp.where(v == t_star, jnp.int32(n - 1) - ii, jnp.int32(-1))
            sel = (v > t_star) | (k2 >= t2)
            val = jnp.where(sel, jnp.float32(1.0), jnp.float32(0.0))
            plsc.store_scatter(out_v, [ii], val, mask=valid)
            return 0
        lax.fori_loop(0, ncsl, f_step, 0)
        pltpu.sync_copy(out_v, out_hbm.at[row])
        return 0

    lax.fori_loop(0, rpw, do_row, 0)


@functools.lru_cache(maxsize=None)
def _make_sc_topk_mask(nrows, n, base=0):
    assert nrows % (_NC * _NS) == 0
    rpw = nrows // (_NC * _NS)
    mesh = plsc.VectorSubcoreMesh(core_axis_name="c", subcore_axis_name="s",
                                  num_cores=_NC, num_subcores=_NS)
    return pl.kernel(
        functools.partial(_sc_body, n, rpw, base),
        out_type=jax.ShapeDtypeStruct((nrows, n), jnp.float32),
        mesh=mesh,
        scratch_types=[
            pltpu.VMEM((n,), jnp.float32),
            pltpu.VMEM((n,), jnp.int32),
            pltpu.VMEM((n,), jnp.int32),
            pltpu.VMEM((n,), jnp.int32),
            pltpu.VMEM((n // _L,), jnp.int32),
            pltpu.VMEM((n,), jnp.float32),
        ],
        compiler_params=pltpu.CompilerParams(needs_layout_passes=False),
    )


_SC_ROWS = 32


def kernel(ck):
    nrows = ck.shape[0]
    sc_rows = _SC_ROWS if nrows > _SC_ROWS else nrows
    if sc_rows == nrows:
        return _make_sc_topk_mask(nrows, ck.shape[1])(ck)
    tc_rows = nrows - sc_rows
    sc_out = _make_sc_topk_mask(sc_rows, ck.shape[1], base=tc_rows)(ck)
    tc_out = _tc_kernel(ck, nrows_out=tc_rows)
    return lax.dynamic_update_slice(tc_out, sc_out, (tc_rows, 0))
